# baseline (speedup 1.0000x reference)
import functools

import jax
import jax.numpy as jnp
from jax import lax
from jax.experimental import pallas as pl
from jax.experimental.pallas import tpu as pltpu
from jax.experimental.pallas import tpu_sc as plsc


def kernel(user_idx, item_idx, user_embedding, item_embedding, user_bias, item_bias):
    B = user_idx.shape[0]
    NU, D = user_embedding.shape
    NI = item_embedding.shape[0]
    info = plsc.get_sparse_core_info()
    NC, NS, L = info.num_cores, info.num_subcores, info.num_lanes
    NW = NC * NS
    assert B % (8 * NW) == 0 and D == L
    bpw = B // NW
    ngrp = bpw // L

    MAXOFF = ((NU - 128) // 128) * 128
    TAIL0 = MAXOFF + 128
    TAILC = TAIL0 // 128
    NTAIL = NU - TAIL0
    ntail_rows = max(NTAIL, 1)

    NSLOT = bpw + 4 * L

    perm = jnp.argsort(user_idx)
    uT = user_embedding.T
    item_flat = item_embedding.reshape(-1)
    ub = user_bias.reshape(-1)
    ib = item_bias.reshape(-1)
    tail_flat = user_embedding[TAIL0:].reshape(-1) if NTAIL else jnp.zeros(
        (16 * D,), jnp.float32)

    mesh = plsc.VectorSubcoreMesh(core_axis_name="c", subcore_axis_name="s")

    @functools.partial(
        pl.kernel,
        mesh=mesh,
        out_type=jax.ShapeDtypeStruct((B,), jnp.float32),
        compiler_params=pltpu.CompilerParams(
            needs_layout_passes=False, use_tc_tiling_on_sc=True),
        scratch_types=[
            pltpu.VMEM((bpw,), jnp.int32),
            pltpu.VMEM((bpw,), jnp.int32),
            pltpu.VMEM((bpw,), jnp.int32),
            pltpu.VMEM((bpw,), jnp.int32),
            pltpu.VMEM((bpw,), jnp.int32),
            pltpu.VMEM((bpw,), jnp.int32),
            pltpu.VMEM((NSLOT,), jnp.int32),
            pltpu.VMEM((2 * L, D, 128), jnp.float32),
            pltpu.VMEM((D * bpw,), jnp.int32),
            pltpu.VMEM((D * bpw,), jnp.float32),
            pltpu.VMEM((ntail_rows * 16,), jnp.float32),
            pltpu.VMEM((bpw,), jnp.float32),
            pltpu.VMEM((bpw,), jnp.float32),
            pltpu.VMEM((bpw,), jnp.float32),
            pltpu.SemaphoreType.DMA,
            pltpu.SemaphoreType.DMA,
        ],
    )
    def run(uidx_h, iidx_h, perm_h, uT_h, itf_h, ub_h, ib_h, tail_h,
            out_h, perm_v, us_v, is_v, cs_v, col_v, s_v, cos_v, ring_v,
            isf_v, ir_v, tail_v, ub_v, ib_v, res_v, sem_ring, sem_misc):
        wid = lax.axis_index("s") * NC + lax.axis_index("c")
        base = wid * bpw
        lane = lax.iota(jnp.int32, L)

        pltpu.sync_copy(perm_h.at[pl.ds(base, bpw)], perm_v)
        cu = pltpu.async_copy(uidx_h.at[perm_v], us_v, sem_misc)
        ci = pltpu.async_copy(iidx_h.at[perm_v], is_v, sem_misc)
        ct = pltpu.async_copy(tail_h, tail_v, sem_misc)
        cu.wait()
        ci.wait()

        cub = pltpu.async_copy(ub_h.at[us_v], ub_v, sem_misc)
        cib = pltpu.async_copy(ib_h.at[is_v], ib_v, sem_misc)

        def p0a(g, carry):
            j0 = g * L
            u = us_v[pl.ds(j0, L)]
            c = u >> 7
            off = jnp.minimum(c * 128, MAXOFF)
            cs_v[pl.ds(j0, L)] = c
            col_v[pl.ds(j0, L)] = u - off
            iv = is_v[pl.ds(j0, L)] * D
            for f in range(D):
                isf_v[pl.ds(f * bpw + j0, L)] = iv + f
            return carry
        lax.fori_loop(0, ngrp, p0a, 0)

        item_copies = [
            pltpu.async_copy(itf_h.at[isf_v.at[pl.ds(f * bpw, bpw)]],
                             ir_v.at[pl.ds(f * bpw, bpw)], sem_misc)
            for f in range(D)
        ]

        def init_md(g, carry):
            cos_v[pl.ds(g * L, L)] = jnp.zeros((L,), jnp.int32)
            return carry
        lax.fori_loop(0, NSLOT // L, init_md, 0)

        def p0b(g, nslots):
            j0 = g * L
            jj = lane + j0
            cur = cs_v[pl.ds(j0, L)]
            prev = plsc.load_gather(cs_v, [jnp.maximum(jj - 1, 0)])
            nf = jnp.logical_or(cur != prev, jj == 0)
            nfi = nf.astype(jnp.int32)
            s = plsc.cumsum(nfi) - 1 + nslots
            s_v[pl.ds(j0, L)] = s
            plsc.store_scatter(cos_v, [s], cur, mask=nf)
            return nslots + lax.reduce_sum_p.bind(nfi, axes=(0,))
        nslots = lax.fori_loop(0, ngrp, p0b, 0)
        ngroups = (nslots + L - 1) // L

        def fire_group(g):
            cvec = cos_v[pl.ds(g * L, L)]
            buf = (g % 2) * L
            for kk in range(L):
                c = cvec[kk]
                off = pl.multiple_of(jnp.minimum(c * 128, MAXOFF), 128)
                pltpu.async_copy(uT_h.at[:, pl.ds(off, 128)],
                                 ring_v.at[buf + kk], sem_ring)

        def wait_group():
            for kk in range(L):
                pltpu.make_async_copy(uT_h.at[:, pl.ds(0, 128)],
                                      ring_v.at[kk], sem_ring).wait()

        fire_group(0)
        ct.wait()
        cub.wait()
        cib.wait()
        for c in item_copies:
            c.wait()

        def process_group(pg):
            j0 = pg * L
            jj = lane + j0
            colv = col_v[pl.ds(j0, L)]
            cvec = cs_v[pl.ds(j0, L)]
            svec = s_v[pl.ds(j0, L)] & (2 * L - 1)
            colc = jnp.minimum(colv, 127)
            tbase = jnp.clip(colv - 128, 0, ntail_rows - 1) * D
            istail = cvec >= TAILC
            acc = ub_v[pl.ds(j0, L)] + ib_v[pl.ds(j0, L)]
            for f in range(D):
                fb = jnp.full((L,), f, jnp.int32)
                uf = plsc.load_gather(ring_v, [svec, fb, colc])
                tf = plsc.load_gather(tail_v, [tbase + f])
                uf = jnp.where(istail, tf, uf)
                itf = plsc.load_gather(ir_v, [jj + f * bpw])
                acc = acc + uf * itf
            res_v[pl.ds(j0, L)] = acc

        def p1(g, pg):
            wait_group()
            def cond(pg2):
                inb = pg2 * L < bpw
                slast = plsc.load_gather(
                    s_v, [jnp.full((L,), jnp.minimum(pg2 * L + L - 1, bpw - 1),
                                   jnp.int32)])[0]
                return jnp.logical_and(inb, slast < (g + 1) * L)
            def body(pg2):
                process_group(pg2)
                return pg2 + 1
            pg = lax.while_loop(cond, body, pg)
            fire_group(g + 1)
            return pg
        lax.fori_loop(0, ngroups, p1, 0)
        wait_group()

        pltpu.async_copy(res_v, out_h.at[perm_v], sem_misc).wait()

    return run(user_idx, item_idx, perm, uT, item_flat, ub, ib, tail_flat)


# speedup vs baseline: 1.2041x; 1.2041x over previous
import functools

import jax
import jax.numpy as jnp
from jax import lax
from jax.experimental import pallas as pl
from jax.experimental.pallas import tpu as pltpu
from jax.experimental.pallas import tpu_sc as plsc


def kernel(user_idx, item_idx, user_embedding, item_embedding, user_bias, item_bias):
    B = user_idx.shape[0]
    NU, D = user_embedding.shape
    NI = item_embedding.shape[0]
    info = plsc.get_sparse_core_info()
    NC, NS, L = info.num_cores, info.num_subcores, info.num_lanes
    NW = NC * NS
    assert B % (8 * NW) == 0 and D == L and NI % 8 == 0
    bpw = B // NW
    ngrp = bpw // L

    MAXOFF = ((NU - 128) // 128) * 128
    TAIL0 = MAXOFF + 128
    TAILC = TAIL0 // 128
    NTAIL = NU - TAIL0
    IBOFF = ((NI - 128) // 128) * 128
    IBT0 = IBOFF + 128
    IBTAILN = NI - IBT0

    NSLOT = bpw + 4 * L

    perm = jnp.argsort(user_idx)
    uT = user_embedding.T
    ubT = user_bias.T
    ibT = item_bias.T

    mesh = plsc.VectorSubcoreMesh(core_axis_name="c", subcore_axis_name="s")

    @functools.partial(
        pl.kernel,
        mesh=mesh,
        out_type=jax.ShapeDtypeStruct((B,), jnp.float32),
        compiler_params=pltpu.CompilerParams(
            needs_layout_passes=False, use_tc_tiling_on_sc=True),
        scratch_types=[
            pltpu.VMEM((bpw,), jnp.int32),
            pltpu.VMEM((bpw,), jnp.int32),
            pltpu.VMEM((bpw,), jnp.int32),
            pltpu.VMEM((bpw,), jnp.int32),
            pltpu.VMEM((bpw,), jnp.int32),
            pltpu.VMEM((bpw,), jnp.int32),
            pltpu.VMEM((NSLOT,), jnp.int32),
            pltpu.VMEM((2 * L, D, 128), jnp.float32),
            pltpu.VMEM((2 * L, 1, 128), jnp.float32),
            pltpu.VMEM((2 * L, 8, D), jnp.float32),
            pltpu.VMEM((2 * L, 1, 128), jnp.float32),
            pltpu.VMEM((bpw * D,), jnp.float32),
            pltpu.VMEM((bpw,), jnp.float32),
            pltpu.VMEM((D, max(NTAIL, 1)), jnp.float32),
            pltpu.VMEM((1, max(NTAIL, 1)), jnp.float32),
            pltpu.VMEM((1, max(IBTAILN, 1)), jnp.float32),
            pltpu.VMEM((bpw,), jnp.float32),
            pltpu.SemaphoreType.DMA,
            pltpu.SemaphoreType.DMA,
            pltpu.SemaphoreType.DMA,
        ],
    )
    def run(uidx_h, iidx_h, perm_h, uT_h, item_h, ubT_h, ibT_h,
            out_h, perm_v, us_v, is_v, cs_v, col_v, s_v, cos_v,
            uring, ubring, iring, ibring, ir_v, ib_v,
            tailu_v, tailb_v, ibtail_v, res_v,
            sem_ring, sem_item, sem_misc):
        wid = lax.axis_index("s") * NC + lax.axis_index("c")
        base = wid * bpw
        lane = lax.iota(jnp.int32, L)
        zl = jnp.zeros((L,), jnp.int32)

        pltpu.sync_copy(perm_h.at[pl.ds(base, bpw)], perm_v)
        cu = pltpu.async_copy(uidx_h.at[perm_v], us_v, sem_misc)
        ci = pltpu.async_copy(iidx_h.at[perm_v], is_v, sem_misc)
        if NTAIL:
            ctu = pltpu.async_copy(uT_h.at[:, pl.ds(TAIL0, NTAIL)], tailu_v,
                                   sem_misc)
            ctb = pltpu.async_copy(ubT_h.at[:, pl.ds(TAIL0, NTAIL)], tailb_v,
                                   sem_misc)
        if IBTAILN:
            cti = pltpu.async_copy(ibT_h.at[:, pl.ds(IBT0, IBTAILN)], ibtail_v,
                                   sem_misc)
        cu.wait()
        ci.wait()
        if NTAIL:
            ctu.wait()
            ctb.wait()
        if IBTAILN:
            cti.wait()

        def fire_item_group(g2):
            ivec = is_v[pl.ds(g2 * L, L)]
            buf = (g2 % 2) * L
            for kk in range(L):
                i = ivec[kk]
                rg8 = pl.multiple_of((i >> 3) << 3, 8)
                pltpu.async_copy(item_h.at[pl.ds(rg8, 8), :],
                                 iring.at[buf + kk], sem_item)
                ibo = pl.multiple_of(
                    jnp.minimum((i >> 7) << 7, IBOFF), 128)
                pltpu.async_copy(ibT_h.at[:, pl.ds(ibo, 128)],
                                 ibring.at[buf + kk], sem_item)

        def wait_item_group():
            for kk in range(L):
                pltpu.make_async_copy(item_h.at[pl.ds(0, 8), :],
                                      iring.at[kk], sem_item).wait()
                pltpu.make_async_copy(ibT_h.at[:, pl.ds(0, 128)],
                                      ibring.at[kk], sem_item).wait()

        fire_item_group(0)

        def pA(g2, carry):

            @pl.when(g2 < ngrp - 1)
            def _():
                fire_item_group(g2 + 1)

            wait_item_group()
            j0 = g2 * L
            ivec = is_v[pl.ds(j0, L)]
            buf = (g2 % 2) * L
            for kk in range(L):
                i = ivec[kk]
                vec = plsc.load_gather(
                    iring, [jnp.full((L,), buf + kk, jnp.int32),
                            jnp.full((L,), i & 7, jnp.int32), lane])
                ir_v[pl.ds((j0 + kk) * D, L)] = vec
            slotv = jnp.full((L,), buf, jnp.int32) + lane
            ibo = jnp.minimum((ivec >> 7) << 7, IBOFF)
            ibcol = jnp.minimum(ivec - ibo, 127)
            ibv = plsc.load_gather(ibring, [slotv, zl, ibcol])
            if IBTAILN:
                tb = plsc.load_gather(
                    ibtail_v, [zl, jnp.clip(ivec - IBT0, 0, IBTAILN - 1)])
                ibv = jnp.where(ivec >= IBT0, tb, ibv)
            ib_v[pl.ds(j0, L)] = ibv
            return carry
        lax.fori_loop(0, ngrp, pA, 0)

        def p0a(g, carry):
            j0 = g * L
            u = us_v[pl.ds(j0, L)]
            c = u >> 7
            off = jnp.minimum(c * 128, MAXOFF)
            cs_v[pl.ds(j0, L)] = c
            col_v[pl.ds(j0, L)] = u - off
            return carry
        lax.fori_loop(0, ngrp, p0a, 0)

        def init_md(g, carry):
            cos_v[pl.ds(g * L, L)] = jnp.zeros((L,), jnp.int32)
            return carry
        lax.fori_loop(0, NSLOT // L, init_md, 0)

        def p0b(g, nslots):
            j0 = g * L
            jj = lane + j0
            cur = cs_v[pl.ds(j0, L)]
            prev = plsc.load_gather(cs_v, [jnp.maximum(jj - 1, 0)])
            nf = jnp.logical_or(cur != prev, jj == 0)
            nfi = nf.astype(jnp.int32)
            s = plsc.cumsum(nfi) - 1 + nslots
            s_v[pl.ds(j0, L)] = s
            plsc.store_scatter(cos_v, [s], cur, mask=nf)
            return nslots + lax.reduce_sum_p.bind(nfi, axes=(0,))
        nslots = lax.fori_loop(0, ngrp, p0b, 0)
        ngroups = (nslots + L - 1) // L

        def fire_group(g):
            cvec = cos_v[pl.ds(g * L, L)]
            buf = (g % 2) * L
            for kk in range(L):
                c = cvec[kk]
                off = pl.multiple_of(jnp.minimum(c * 128, MAXOFF), 128)
                pltpu.async_copy(uT_h.at[:, pl.ds(off, 128)],
                                 uring.at[buf + kk], sem_ring)
                pltpu.async_copy(ubT_h.at[:, pl.ds(off, 128)],
                                 ubring.at[buf + kk], sem_ring)

        def wait_group():
            for kk in range(L):
                pltpu.make_async_copy(uT_h.at[:, pl.ds(0, 128)],
                                      uring.at[kk], sem_ring).wait()
                pltpu.make_async_copy(ubT_h.at[:, pl.ds(0, 128)],
                                      ubring.at[kk], sem_ring).wait()

        fire_group(0)

        def process_group(pg):
            j0 = pg * L
            jj = lane + j0
            colv = col_v[pl.ds(j0, L)]
            cvec = cs_v[pl.ds(j0, L)]
            svec = s_v[pl.ds(j0, L)] & (2 * L - 1)
            colc = jnp.minimum(colv, 127)
            istail = cvec >= TAILC
            coltv = jnp.clip(colv - 128, 0, max(NTAIL, 1) - 1)
            ubv = plsc.load_gather(ubring, [svec, zl, colc])
            if NTAIL:
                ubt = plsc.load_gather(tailb_v, [zl, coltv])
                ubv = jnp.where(istail, ubt, ubv)
            acc = ubv + ib_v[pl.ds(j0, L)]
            jd = jj * D
            for f in range(D):
                fb = jnp.full((L,), f, jnp.int32)
                uf = plsc.load_gather(uring, [svec, fb, colc])
                if NTAIL:
                    tf = plsc.load_gather(tailu_v, [fb, coltv])
                    uf = jnp.where(istail, tf, uf)
                itf = plsc.load_gather(ir_v, [jd + f])
                acc = acc + uf * itf
            res_v[pl.ds(j0, L)] = acc

        def p1(g, pg):
            wait_group()
            def cond(pg2):
                inb = pg2 * L < bpw
                slast = plsc.load_gather(
                    s_v, [jnp.full((L,), jnp.minimum(pg2 * L + L - 1, bpw - 1),
                                   jnp.int32)])[0]
                return jnp.logical_and(inb, slast < (g + 1) * L)
            def body(pg2):
                process_group(pg2)
                return pg2 + 1
            pg = lax.while_loop(cond, body, pg)
            fire_group(g + 1)
            return pg
        lax.fori_loop(0, ngroups, p1, 0)
        wait_group()

        pltpu.async_copy(res_v, out_h.at[perm_v], sem_misc).wait()

    return run(user_idx, item_idx, perm, uT, item_embedding, ubT, ibT)


# speedup vs baseline: 1.2715x; 1.0560x over previous
import functools

import jax
import jax.numpy as jnp
from jax import lax
from jax.experimental import pallas as pl
from jax.experimental.pallas import tpu as pltpu
from jax.experimental.pallas import tpu_sc as plsc


def kernel(user_idx, item_idx, user_embedding, item_embedding, user_bias, item_bias):
    B = user_idx.shape[0]
    NU, D = user_embedding.shape
    NI = item_embedding.shape[0]
    info = plsc.get_sparse_core_info()
    NC, NS, L = info.num_cores, info.num_subcores, info.num_lanes
    NW = NC * NS
    assert B % (8 * NW) == 0 and D == L and NI % 8 == 0
    bpw = B // NW
    ngrp = bpw // L

    MAXOFF = ((NU - 128) // 128) * 128
    TAIL0 = MAXOFF + 128
    TAILC = TAIL0 // 128
    NTAIL = NU - TAIL0
    IBOFF = ((NI - 128) // 128) * 128
    IBT0 = IBOFF + 128
    IBTAILN = NI - IBT0

    NSLOT = bpw + 4 * L

    perm = jnp.argsort(user_idx)
    uT = user_embedding.T
    ubT = user_bias.T
    ibT = item_bias.T

    mesh = plsc.VectorSubcoreMesh(core_axis_name="c", subcore_axis_name="s")

    @functools.partial(
        pl.kernel,
        mesh=mesh,
        out_type=jax.ShapeDtypeStruct((B,), jnp.float32),
        compiler_params=pltpu.CompilerParams(
            needs_layout_passes=False, use_tc_tiling_on_sc=True),
        scratch_types=[
            pltpu.VMEM((bpw,), jnp.int32),
            pltpu.VMEM((bpw,), jnp.int32),
            pltpu.VMEM((bpw,), jnp.int32),
            pltpu.VMEM((bpw,), jnp.int32),
            pltpu.VMEM((bpw,), jnp.int32),
            pltpu.VMEM((bpw,), jnp.int32),
            pltpu.VMEM((NSLOT,), jnp.int32),
            pltpu.VMEM((32, D, 128), jnp.float32),
            pltpu.VMEM((32, 1, 128), jnp.float32),
            pltpu.VMEM((2 * L, 8, D), jnp.float32),
            pltpu.VMEM((2 * L, 1, 128), jnp.float32),
            pltpu.VMEM((bpw * D,), jnp.float32),
            pltpu.VMEM((bpw,), jnp.float32),
            pltpu.VMEM((D, max(NTAIL, 1)), jnp.float32),
            pltpu.VMEM((1, max(NTAIL, 1)), jnp.float32),
            pltpu.VMEM((1, max(IBTAILN, 1)), jnp.float32),
            pltpu.VMEM((bpw,), jnp.float32),
            pltpu.SemaphoreType.DMA,
            pltpu.SemaphoreType.DMA,
            pltpu.SemaphoreType.DMA,
        ],
    )
    def run(uidx_h, iidx_h, perm_h, uT_h, item_h, ubT_h, ibT_h,
            out_h, perm_v, us_v, is_v, cs_v, col_v, s_v, cos_v,
            uring, ubring, iring, ibring, ir_v, ib_v,
            tailu_v, tailb_v, ibtail_v, res_v,
            sem_ring, sem_item, sem_misc):
        wid = lax.axis_index("s") * NC + lax.axis_index("c")
        base = wid * bpw
        lane = lax.iota(jnp.int32, L)
        zl = jnp.zeros((L,), jnp.int32)

        pltpu.sync_copy(perm_h.at[pl.ds(base, bpw)], perm_v)
        cu = pltpu.async_copy(uidx_h.at[perm_v], us_v, sem_misc)
        ci = pltpu.async_copy(iidx_h.at[perm_v], is_v, sem_misc)
        if NTAIL:
            ctu = pltpu.async_copy(uT_h.at[:, pl.ds(TAIL0, NTAIL)], tailu_v,
                                   sem_misc)
            ctb = pltpu.async_copy(ubT_h.at[:, pl.ds(TAIL0, NTAIL)], tailb_v,
                                   sem_misc)
        if IBTAILN:
            cti = pltpu.async_copy(ibT_h.at[:, pl.ds(IBT0, IBTAILN)], ibtail_v,
                                   sem_misc)
        cu.wait()
        ci.wait()
        if NTAIL:
            ctu.wait()
            ctb.wait()
        if IBTAILN:
            cti.wait()

        def fire_item_group(g2):
            ivec = is_v[pl.ds(g2 * L, L)]
            buf = (g2 % 2) * L
            for kk in range(L):
                i = ivec[kk]
                rg8 = pl.multiple_of((i >> 3) << 3, 8)
                pltpu.async_copy(item_h.at[pl.ds(rg8, 8), :],
                                 iring.at[buf + kk], sem_item)
                ibo = pl.multiple_of(
                    jnp.minimum((i >> 7) << 7, IBOFF), 128)
                pltpu.async_copy(ibT_h.at[:, pl.ds(ibo, 128)],
                                 ibring.at[buf + kk], sem_item)

        def wait_item_group():
            for kk in range(L):
                pltpu.make_async_copy(item_h.at[pl.ds(0, 8), :],
                                      iring.at[kk], sem_item).wait()
                pltpu.make_async_copy(ibT_h.at[:, pl.ds(0, 128)],
                                      ibring.at[kk], sem_item).wait()

        fire_item_group(0)

        def pA(g2, carry):

            @pl.when(g2 < ngrp - 1)
            def _():
                fire_item_group(g2 + 1)

            wait_item_group()
            j0 = g2 * L
            ivec = is_v[pl.ds(j0, L)]
            buf = (g2 % 2) * L
            for kk in range(L):
                i = ivec[kk]
                vec = plsc.load_gather(
                    iring, [jnp.full((L,), buf + kk, jnp.int32),
                            jnp.full((L,), i & 7, jnp.int32), lane])
                ir_v[pl.ds((j0 + kk) * D, L)] = vec
            slotv = jnp.full((L,), buf, jnp.int32) + lane
            ibo = jnp.minimum((ivec >> 7) << 7, IBOFF)
            ibcol = jnp.minimum(ivec - ibo, 127)
            ibv = plsc.load_gather(ibring, [slotv, zl, ibcol])
            if IBTAILN:
                tb = plsc.load_gather(
                    ibtail_v, [zl, jnp.clip(ivec - IBT0, 0, IBTAILN - 1)])
                ibv = jnp.where(ivec >= IBT0, tb, ibv)
            ib_v[pl.ds(j0, L)] = ibv
            return carry
        lax.fori_loop(0, ngrp, pA, 0)

        def p0a(g, carry):
            j0 = g * L
            u = us_v[pl.ds(j0, L)]
            c = u >> 7
            off = jnp.minimum(c * 128, MAXOFF)
            cs_v[pl.ds(j0, L)] = c
            col_v[pl.ds(j0, L)] = u - off
            return carry
        lax.fori_loop(0, ngrp, p0a, 0)

        def init_md(g, carry):
            cos_v[pl.ds(g * L, L)] = jnp.zeros((L,), jnp.int32)
            return carry
        lax.fori_loop(0, NSLOT // L, init_md, 0)

        def p0b(g, nslots):
            j0 = g * L
            jj = lane + j0
            cur = cs_v[pl.ds(j0, L)]
            prev = plsc.load_gather(cs_v, [jnp.maximum(jj - 1, 0)])
            nf = jnp.logical_or(cur != prev, jj == 0)
            nfi = nf.astype(jnp.int32)
            s = plsc.cumsum(nfi) - 1 + nslots
            s_v[pl.ds(j0, L)] = s
            plsc.store_scatter(cos_v, [s], cur, mask=nf)
            return nslots + lax.reduce_sum_p.bind(nfi, axes=(0,))
        nslots = lax.fori_loop(0, ngrp, p0b, 0)
        ngroups = (nslots + 8 - 1) // 8

        GB = 8
        def fire_group(g):
            cvec = cos_v[pl.ds(g * GB, L)]
            buf = (g % 4) * GB
            for kk in range(GB):
                c = cvec[kk]
                off = pl.multiple_of(jnp.minimum(c * 128, MAXOFF), 128)
                pltpu.async_copy(uT_h.at[:, pl.ds(off, 128)],
                                 uring.at[buf + kk], sem_ring)
                pltpu.async_copy(ubT_h.at[:, pl.ds(off, 128)],
                                 ubring.at[buf + kk], sem_ring)

        def wait_group():
            for kk in range(GB):
                pltpu.make_async_copy(uT_h.at[:, pl.ds(0, 128)],
                                      uring.at[kk], sem_ring).wait()
                pltpu.make_async_copy(ubT_h.at[:, pl.ds(0, 128)],
                                      ubring.at[kk], sem_ring).wait()

        fire_group(0)
        fire_group(1)

        def process_group(pg):
            j0 = pg * L
            jj = lane + j0
            colv = col_v[pl.ds(j0, L)]
            cvec = cs_v[pl.ds(j0, L)]
            svec = s_v[pl.ds(j0, L)] % 32
            colc = jnp.minimum(colv, 127)
            istail = cvec >= TAILC
            coltv = jnp.clip(colv - 128, 0, max(NTAIL, 1) - 1)
            ubv = plsc.load_gather(ubring, [svec, zl, colc])
            if NTAIL:
                ubt = plsc.load_gather(tailb_v, [zl, coltv])
                ubv = jnp.where(istail, ubt, ubv)
            acc = ubv + ib_v[pl.ds(j0, L)]
            jd = jj * D
            for f in range(D):
                fb = jnp.full((L,), f, jnp.int32)
                uf = plsc.load_gather(uring, [svec, fb, colc])
                if NTAIL:
                    tf = plsc.load_gather(tailu_v, [fb, coltv])
                    uf = jnp.where(istail, tf, uf)
                itf = plsc.load_gather(ir_v, [jd + f])
                acc = acc + uf * itf
            res_v[pl.ds(j0, L)] = acc

        def p1(g, pg):
            wait_group()
            def cond(pg2):
                inb = pg2 * L < bpw
                slast = plsc.load_gather(
                    s_v, [jnp.full((L,), jnp.minimum(pg2 * L + L - 1, bpw - 1),
                                   jnp.int32)])[0]
                return jnp.logical_and(inb, slast < (g + 1) * GB)
            def body(pg2):
                process_group(pg2)
                return pg2 + 1
            pg = lax.while_loop(cond, body, pg)
            fire_group(g + 2)
            return pg
        lax.fori_loop(0, ngroups, p1, 0)
        wait_group()
        wait_group()

        pltpu.async_copy(res_v, out_h.at[perm_v], sem_misc).wait()

    return run(user_idx, item_idx, perm, uT, item_embedding, ubT, ibT)
